# baseline (speedup 1.0000x reference)
import functools

import jax
import jax.numpy as jnp
from jax import lax
from jax.experimental import pallas as pl
from jax.experimental.pallas import tpu as pltpu
from jax.experimental.pallas import tpu_sc as plsc

PAD_ID_K = 0
D_K = 64
SCALE_K = float(D_K) ** 0.5

NC_K = 2
NS_K = 16
NW_K = NC_K * NS_K
NB_K = 4096
NJ_K = 200
V_K = 1000000
BW_K = NB_K // NW_K
NBUF_K = 4


def _emb_body(ids_hbm, tab_hbm, out_hbm, ids_v, gbuf, obuf, sem_g, sem_o):
    wid = lax.axis_index("s") * NC_K + lax.axis_index("c")
    b0 = wid * BW_K
    pltpu.sync_copy(ids_hbm.at[:, pl.ds(b0, BW_K)], ids_v)

    iota = lax.iota(jnp.int32, 16)

    def pre_body(j, c):
        for g in range(BW_K // 16):
            iv = ids_v[j, pl.ds(g * 16, 16)]
            ids_v[j, pl.ds(g * 16, 16)] = jnp.where(iv != PAD_ID_K, iv + iv, 1)
        return c

    lax.fori_loop(0, NJ_K, pre_body, 0)

    def gather_cp(j, bb):
        return pltpu.make_async_copy(
            tab_hbm.at[ids_v.at[j]], gbuf.at[bb], sem_g.at[bb]
        )

    def scatter_cp(j, bb):
        return pltpu.make_async_copy(
            obuf.at[bb], out_hbm.at[j, :, wid], sem_o.at[bb]
        )

    def compute(j, bb):
        bbv = jnp.zeros((16,), jnp.int32) + bb

        def blk_body(blk, c):
            bvec = jnp.zeros((16,), jnp.int32) + ((blk & 7) * 16) + iota
            cbase = jnp.zeros((16,), jnp.int32) + ((blk >> 3) * 16)
            for d in range(16):
                cvec = cbase + ((iota + d) & 15)
                val = plsc.load_gather(gbuf, [bbv, bvec, cvec])
                plsc.store_scatter(
                    obuf,
                    [bbv, lax.shift_right_logical(cvec, 3), cvec & 7, bvec],
                    val,
                )
            return c

        lax.fori_loop(0, (BW_K // 16) * (D_K // 16), blk_body, 0)

    for bb in range(NBUF_K):
        gather_cp(bb, bb).start()
    for bb in range(NBUF_K):
        gather_cp(bb, bb).wait()
        compute(bb, bb)
        scatter_cp(bb, bb).start()
        gather_cp(bb + NBUF_K, bb).start()

    def quad_body(p, carry):
        for bb in range(NBUF_K):
            j = p * NBUF_K + bb
            gather_cp(j, bb).wait()
            scatter_cp(j - NBUF_K, bb).wait()
            compute(j, bb)
            scatter_cp(j, bb).start()
            gather_cp(j + NBUF_K, bb).start()
        return carry

    lax.fori_loop(1, NJ_K // NBUF_K - 1, quad_body, 0)

    for bb in range(NBUF_K):
        j = NJ_K - NBUF_K + bb
        gather_cp(j, bb).wait()
        scatter_cp(j - NBUF_K, bb).wait()
        compute(j, bb)
        scatter_cp(j, bb).start()
    for bb in range(NBUF_K):
        scatter_cp(NJ_K - NBUF_K + bb, bb).wait()


_emb = functools.partial(
    pl.kernel,
    out_type=jax.ShapeDtypeStruct((NJ_K, 8, NW_K, 8, 128), jnp.float32),
    mesh=plsc.VectorSubcoreMesh(core_axis_name="c", subcore_axis_name="s"),
    scratch_types=[
        pltpu.VMEM((NJ_K, BW_K), jnp.int32),
        pltpu.VMEM((NBUF_K, BW_K, D_K), jnp.float32),
        pltpu.VMEM((NBUF_K, 8, 8, 128), jnp.float32),
        pltpu.SemaphoreType.DMA((NBUF_K,)),
        pltpu.SemaphoreType.DMA((NBUF_K,)),
    ],
    compiler_params=pltpu.CompilerParams(
        needs_layout_passes=False, use_tc_tiling_on_sc=False
    ),
)(_emb_body)


@jax.jit
def kernel(input, lookup_table):
    ids_t = input.T.astype(jnp.int32)
    tabp = jnp.pad(lookup_table * jnp.float32(SCALE_K), ((0, 0), (0, D_K)))
    tab2 = tabp.reshape(2 * V_K, D_K)
    out5 = _emb(ids_t, tab2)
    return out5.transpose(2, 4, 0, 1, 3).reshape(NB_K, NJ_K, D_K)


# speedup vs baseline: 1.0189x; 1.0189x over previous
import functools

import jax
import jax.numpy as jnp
from jax import lax
from jax.experimental import pallas as pl
from jax.experimental.pallas import tpu as pltpu
from jax.experimental.pallas import tpu_sc as plsc

PAD_ID_K = 0
D_K = 64
SCALE_K = float(D_K) ** 0.5

NC_K = 2
NS_K = 16
NW_K = NC_K * NS_K
NB_K = 4096
NJ_K = 200
V_K = 1000000
BW_K = NB_K // NW_K
NBUF_K = 4


def _emb_body(ids_hbm, tab_hbm, out_hbm, ids_v, gbuf, obuf, sem_g, sem_o):
    wid = lax.axis_index("s") * NC_K + lax.axis_index("c")
    b0 = wid * BW_K
    pltpu.sync_copy(ids_hbm.at[:, pl.ds(b0, BW_K)], ids_v)

    iota = lax.iota(jnp.int32, 16)

    def pre_body(j, c):
        for g in range(BW_K // 16):
            iv = ids_v[j, pl.ds(g * 16, 16)]
            ids_v[j, pl.ds(g * 16, 16)] = jnp.where(iv != PAD_ID_K, iv + iv, 1)
        return c

    lax.fori_loop(0, NJ_K, pre_body, 0)

    def gather_cp(j, bb):
        return pltpu.make_async_copy(
            tab_hbm.at[ids_v.at[j]], gbuf.at[bb], sem_g.at[bb]
        )

    def scatter_cp(j, bb):
        return pltpu.make_async_copy(
            obuf.at[bb], out_hbm.at[j, :, wid], sem_o.at[bb]
        )

    def compute(j, bb):
        bbv = jnp.zeros((16,), jnp.int32) + bb

        def blk_body(blk, c):
            bvec = jnp.zeros((16,), jnp.int32) + ((blk & 7) * 16) + iota
            cbase = jnp.zeros((16,), jnp.int32) + ((blk >> 3) * 16)
            for d in range(16):
                cvec = cbase + ((iota + d) & 15)
                val = plsc.load_gather(gbuf, [bbv, bvec, cvec]) * jnp.float32(SCALE_K)
                plsc.store_scatter(
                    obuf,
                    [bbv, lax.shift_right_logical(cvec, 3), cvec & 7, bvec],
                    val,
                )
            return c

        lax.fori_loop(0, (BW_K // 16) * (D_K // 16), blk_body, 0)

    for bb in range(NBUF_K):
        gather_cp(bb, bb).start()
    for bb in range(NBUF_K):
        gather_cp(bb, bb).wait()
        compute(bb, bb)
        scatter_cp(bb, bb).start()
        gather_cp(bb + NBUF_K, bb).start()

    def quad_body(p, carry):
        for bb in range(NBUF_K):
            j = p * NBUF_K + bb
            gather_cp(j, bb).wait()
            scatter_cp(j - NBUF_K, bb).wait()
            compute(j, bb)
            scatter_cp(j, bb).start()
            gather_cp(j + NBUF_K, bb).start()
        return carry

    lax.fori_loop(1, NJ_K // NBUF_K - 1, quad_body, 0)

    for bb in range(NBUF_K):
        j = NJ_K - NBUF_K + bb
        gather_cp(j, bb).wait()
        scatter_cp(j - NBUF_K, bb).wait()
        compute(j, bb)
        scatter_cp(j, bb).start()
    for bb in range(NBUF_K):
        scatter_cp(NJ_K - NBUF_K + bb, bb).wait()


_emb = functools.partial(
    pl.kernel,
    out_type=jax.ShapeDtypeStruct((NJ_K, 8, NW_K, 8, 128), jnp.float32),
    mesh=plsc.VectorSubcoreMesh(core_axis_name="c", subcore_axis_name="s"),
    scratch_types=[
        pltpu.VMEM((NJ_K, BW_K), jnp.int32),
        pltpu.VMEM((NBUF_K, BW_K, D_K), jnp.float32),
        pltpu.VMEM((NBUF_K, 8, 8, 128), jnp.float32),
        pltpu.SemaphoreType.DMA((NBUF_K,)),
        pltpu.SemaphoreType.DMA((NBUF_K,)),
    ],
    compiler_params=pltpu.CompilerParams(
        needs_layout_passes=False, use_tc_tiling_on_sc=False
    ),
)(_emb_body)


@jax.jit
def kernel(input, lookup_table):
    ids_t = input.T.astype(jnp.int32)
    tabp = jnp.pad(lookup_table, ((0, 0), (0, D_K)))
    tab2 = tabp.reshape(2 * V_K, D_K)
    out5 = _emb(ids_t, tab2)
    return out5.transpose(2, 4, 0, 1, 3).reshape(NB_K, NJ_K, D_K)


# speedup vs baseline: 1.0366x; 1.0173x over previous
import functools

import jax
import jax.numpy as jnp
from jax import lax
from jax.experimental import pallas as pl
from jax.experimental.pallas import tpu as pltpu
from jax.experimental.pallas import tpu_sc as plsc

PAD_ID_K = 0
D_K = 64
SCALE_K = float(D_K) ** 0.5

NC_K = 2
NS_K = 16
NW_K = NC_K * NS_K
NB_K = 4096
NJ_K = 200
V_K = 1000000
BW_K = NB_K // NW_K
NBUF_K = 4
PIT_K = 129


def _emb_body(ids_hbm, tab_hbm, out_hbm, ids_v, gbufs, obufs, sem_g, sem_o):
    wid = lax.axis_index("s") * NC_K + lax.axis_index("c")
    b0 = wid * BW_K
    pltpu.sync_copy(ids_hbm.at[:, pl.ds(b0, BW_K)], ids_v)

    iota = lax.iota(jnp.int32, 16)
    zero16 = jnp.zeros((16,), jnp.int32)
    chi = [lax.shift_right_logical(iota + (g * 16), 3) for g in range(4)]
    clo = [(iota + (g * 16)) & 7 for g in range(4)]

    def pre_body(j, c):
        for g in range(BW_K // 16):
            iv = ids_v[j, pl.ds(g * 16, 16)]
            ids_v[j, pl.ds(g * 16, 16)] = jnp.where(iv != PAD_ID_K, iv + iv, 1)
        return c

    lax.fori_loop(0, NJ_K, pre_body, 0)

    def gather_cp(j, bb):
        return pltpu.make_async_copy(
            tab_hbm.at[ids_v.at[j]], gbufs[bb], sem_g.at[bb]
        )

    def scatter_cp(j, bb):
        return pltpu.make_async_copy(
            obufs[bb].at[:, :, pl.ds(0, 128)], out_hbm.at[j, :, wid], sem_o.at[bb]
        )

    def compute(j, bb):
        gbuf, obuf = gbufs[bb], obufs[bb]

        def row_body(b, c):
            bs = zero16 + b
            for g in range(4):
                v = gbuf[b, pl.ds(g * 16, 16)] * jnp.float32(SCALE_K)
                plsc.store_scatter(obuf, [chi[g], clo[g], bs], v)
            return c

        lax.fori_loop(0, BW_K, row_body, 0)

    for bb in range(NBUF_K):
        gather_cp(bb, bb).start()
    for bb in range(NBUF_K):
        gather_cp(bb, bb).wait()
        compute(bb, bb)
        scatter_cp(bb, bb).start()
        gather_cp(bb + NBUF_K, bb).start()

    def quad_body(p, carry):
        for bb in range(NBUF_K):
            j = p * NBUF_K + bb
            gather_cp(j, bb).wait()
            scatter_cp(j - NBUF_K, bb).wait()
            compute(j, bb)
            scatter_cp(j, bb).start()
            gather_cp(j + NBUF_K, bb).start()
        return carry

    lax.fori_loop(1, NJ_K // NBUF_K - 1, quad_body, 0)

    for bb in range(NBUF_K):
        j = NJ_K - NBUF_K + bb
        gather_cp(j, bb).wait()
        scatter_cp(j - NBUF_K, bb).wait()
        compute(j, bb)
        scatter_cp(j, bb).start()
    for bb in range(NBUF_K):
        scatter_cp(NJ_K - NBUF_K + bb, bb).wait()


def _body_wrap(ids_hbm, tab_hbm, out_hbm, ids_v, g0, g1, g2, g3, o0, o1, o2, o3,
               sem_g, sem_o):
    _emb_body(ids_hbm, tab_hbm, out_hbm, ids_v,
              [g0, g1, g2, g3], [o0, o1, o2, o3], sem_g, sem_o)


_emb = functools.partial(
    pl.kernel,
    out_type=jax.ShapeDtypeStruct((NJ_K, 8, NW_K, 8, 128), jnp.float32),
    mesh=plsc.VectorSubcoreMesh(core_axis_name="c", subcore_axis_name="s"),
    scratch_types=[
        pltpu.VMEM((NJ_K, BW_K), jnp.int32),
    ] + [pltpu.VMEM((BW_K, D_K), jnp.float32) for _ in range(NBUF_K)]
      + [pltpu.VMEM((8, 8, PIT_K), jnp.float32) for _ in range(NBUF_K)]
      + [
        pltpu.SemaphoreType.DMA((NBUF_K,)),
        pltpu.SemaphoreType.DMA((NBUF_K,)),
    ],
    compiler_params=pltpu.CompilerParams(
        needs_layout_passes=False, use_tc_tiling_on_sc=False
    ),
)(_body_wrap)


@jax.jit
def kernel(input, lookup_table):
    ids_t = input.T.astype(jnp.int32)
    tabp = jnp.pad(lookup_table, ((0, 0), (0, D_K)))
    tab2 = tabp.reshape(2 * V_K, D_K)
    out5 = _emb(ids_t, tab2)
    return out5.transpose(2, 4, 0, 1, 3).reshape(NB_K, NJ_K, D_K)


# speedup vs baseline: 1.0384x; 1.0018x over previous
import functools

import jax
import jax.numpy as jnp
from jax import lax
from jax.experimental import pallas as pl
from jax.experimental.pallas import tpu as pltpu
from jax.experimental.pallas import tpu_sc as plsc

PAD_ID_K = 0
D_K = 64
SCALE_K = float(D_K) ** 0.5

NC_K = 2
NS_K = 16
NW_K = NC_K * NS_K
NB_K = 4096
NJ_K = 200
V_K = 1000000
BW_K = NB_K // NW_K
NBUF_K = 5
PIT_K = 129
GS_K = 64


def _emb_body(ids_hbm, tab_hbm, out_hbm, ids_v, gbufs, obufs, sem_g, sem_o):
    wid = lax.axis_index("s") * NC_K + lax.axis_index("c")
    b0 = wid * BW_K
    pltpu.sync_copy(ids_hbm.at[:, pl.ds(b0, BW_K)], ids_v)

    iota = lax.iota(jnp.int32, 16)
    zero16 = jnp.zeros((16,), jnp.int32)
    chi = [lax.shift_right_logical(iota + (g * 16), 3) for g in range(4)]
    clo = [(iota + (g * 16)) & 7 for g in range(4)]

    def pre_body(j, c):
        for g in range(BW_K // 16):
            iv = ids_v[j, pl.ds(g * 16, 16)]
            ids_v[j, pl.ds(g * 16, 16)] = jnp.where(iv != PAD_ID_K, iv + iv, 1)
        return c

    lax.fori_loop(0, NJ_K, pre_body, 0)

    def gather_cps(j, bb):
        return [
            pltpu.make_async_copy(
                tab_hbm.at[ids_v.at[j, pl.ds(h * GS_K, GS_K)]],
                gbufs[bb].at[pl.ds(h * GS_K, GS_K)],
                sem_g.at[bb],
            )
            for h in range(BW_K // GS_K)
        ]

    def gather_start(j, bb):
        for cp in gather_cps(j, bb):
            cp.start()

    def gather_wait(j, bb):
        for cp in gather_cps(j, bb):
            cp.wait()

    def scatter_cp(j, bb):
        return pltpu.make_async_copy(
            obufs[bb].at[:, :, pl.ds(0, 128)], out_hbm.at[j, :, wid], sem_o.at[bb]
        )

    def compute(j, bb):
        gbuf, obuf = gbufs[bb], obufs[bb]

        def row_body(b, c):
            bs = zero16 + b
            for g in range(4):
                v = gbuf[b, pl.ds(g * 16, 16)] * jnp.float32(SCALE_K)
                plsc.store_scatter(obuf, [chi[g], clo[g], bs], v)
            return c

        lax.fori_loop(0, BW_K, row_body, 0)

    for bb in range(NBUF_K):
        gather_start(bb, bb)
    for bb in range(NBUF_K):
        gather_wait(bb, bb)
        compute(bb, bb)
        scatter_cp(bb, bb).start()
        gather_start(bb + NBUF_K, bb)

    def ring_body(p, carry):
        for bb in range(NBUF_K):
            j = p * NBUF_K + bb
            gather_wait(j, bb)
            scatter_cp(j - NBUF_K, bb).wait()
            compute(j, bb)
            scatter_cp(j, bb).start()
            gather_start(j + NBUF_K, bb)
        return carry

    lax.fori_loop(1, NJ_K // NBUF_K - 1, ring_body, 0)

    for bb in range(NBUF_K):
        j = NJ_K - NBUF_K + bb
        gather_wait(j, bb)
        scatter_cp(j - NBUF_K, bb).wait()
        compute(j, bb)
        scatter_cp(j, bb).start()
    for bb in range(NBUF_K):
        scatter_cp(NJ_K - NBUF_K + bb, bb).wait()


def _body_wrap(ids_hbm, tab_hbm, out_hbm, ids_v, g0, g1, g2, g3, g4,
               o0, o1, o2, o3, o4, sem_g, sem_o):
    _emb_body(ids_hbm, tab_hbm, out_hbm, ids_v,
              [g0, g1, g2, g3, g4], [o0, o1, o2, o3, o4], sem_g, sem_o)


_emb = functools.partial(
    pl.kernel,
    out_type=jax.ShapeDtypeStruct((NJ_K, 8, NW_K, 8, 128), jnp.float32),
    mesh=plsc.VectorSubcoreMesh(core_axis_name="c", subcore_axis_name="s"),
    scratch_types=[
        pltpu.VMEM((NJ_K, BW_K), jnp.int32),
    ] + [pltpu.VMEM((BW_K, D_K), jnp.float32) for _ in range(NBUF_K)]
      + [pltpu.VMEM((8, 8, PIT_K), jnp.float32) for _ in range(NBUF_K)]
      + [
        pltpu.SemaphoreType.DMA((NBUF_K,)),
        pltpu.SemaphoreType.DMA((NBUF_K,)),
    ],
    compiler_params=pltpu.CompilerParams(
        needs_layout_passes=False, use_tc_tiling_on_sc=False
    ),
)(_body_wrap)


@jax.jit
def kernel(input, lookup_table):
    ids_t = input.T.astype(jnp.int32)
    tabp = jnp.pad(lookup_table, ((0, 0), (0, D_K)))
    tab2 = tabp.reshape(2 * V_K, D_K)
    out5 = _emb(ids_t, tab2)
    return out5.transpose(2, 4, 0, 1, 3).reshape(NB_K, NJ_K, D_K)
